# baseline (speedup 1.0000x reference)
import functools

import jax
import jax.numpy as jnp
from jax import lax
from jax.experimental import pallas as pl
from jax.experimental.pallas import tpu as pltpu
from jax.experimental.pallas import tpu_sc as plsc

_NUM_CORES = 1
_NUM_SUBCORES = 16
_LANES = 16
_NW = _NUM_CORES * _NUM_SUBCORES
_ROW = 128


@functools.partial(jax.jit, static_argnames=("n", "b"))
def _position_weights(values, offsets, position_weight, n, b):
    chunk = n // _NW
    vecs = chunk // _LANES
    rows = chunk // _ROW
    vecs_per_row = _ROW // _LANES
    mesh = plsc.VectorSubcoreMesh(
        core_axis_name="c", subcore_axis_name="s", num_cores=_NUM_CORES
    )

    @functools.partial(
        pl.kernel,
        mesh=mesh,
        out_type=(
            jax.ShapeDtypeStruct((n,), values.dtype),
            jax.ShapeDtypeStruct((b,), offsets.dtype),
            jax.ShapeDtypeStruct((n,), jnp.float32),
        ),
        compiler_params=pltpu.CompilerParams(needs_layout_passes=False),
        scratch_types=[
            pltpu.VMEM((_LANES,), jnp.int32),
            pltpu.VMEM((n,), jnp.float32),
            pltpu.VMEM((chunk,), jnp.int32),
            pltpu.VMEM((chunk,), jnp.float32),
            pltpu.SemaphoreType.DMA,
            pltpu.SemaphoreType.DMA,
            pltpu.SemaphoreType.DMA,
        ],
    )
    def body(values_hbm, offs_hbm, pw_hbm, vout_hbm, oout_hbm, out_hbm,
             offs_v, pw_v, seq_arr, out_v, tsem, osem, psem):
        wid = lax.axis_index("s") * _NUM_CORES + lax.axis_index("c")
        base = wid * chunk
        table_dma = pltpu.async_copy(pw_hbm, pw_v, tsem)
        pass_dma = pltpu.async_copy(
            values_hbm.at[pl.ds(base, chunk)], vout_hbm.at[pl.ds(base, chunk)], psem
        )

        @pl.when(wid == 0)
        def _():
            pltpu.sync_copy(offs_hbm, oout_hbm)

        pltpu.sync_copy(offs_hbm.at[pl.ds(0, _LANES)], offs_v)

        offs_vec = offs_v[...]
        zero = jnp.zeros((_LANES,), jnp.int32)

        def zero_step(v, carry):
            seq_arr[pl.ds(v * _LANES, _LANES)] = zero
            return carry

        lax.fori_loop(0, vecs, zero_step, 0)
        carry0 = jnp.max(jnp.where(offs_vec <= base, offs_vec, 0))
        in_tile = (offs_vec > base) & (offs_vec < base + chunk)
        plsc.store_scatter(seq_arr, [offs_vec - base], offs_vec, mask=in_tile)

        lane = lax.iota(jnp.int32, _LANES)

        def seq_step(v, carry):
            start = v * _LANES
            off = jnp.maximum(plsc.cummax(seq_arr[pl.ds(start, _LANES)]), carry)
            seq_arr[pl.ds(start, _LANES)] = lane + (base + start) - off
            return off[_LANES - 1]

        lax.fori_loop(0, vecs, seq_step, carry0)

        table_dma.wait()
        out_dmas = []
        for r in range(rows):
            for v in range(vecs_per_row):
                start = r * _ROW + v * _LANES
                out_v[start:start + _LANES] = plsc.load_gather(
                    pw_v, [seq_arr[start:start + _LANES]]
                )
            out_dmas.append(
                pltpu.async_copy(
                    out_v.at[pl.ds(r * _ROW, _ROW)],
                    out_hbm.at[pl.ds(base + r * _ROW, _ROW)],
                    osem,
                )
            )
        for d in out_dmas:
            d.wait()
        pass_dma.wait()

    return body(values, offsets, position_weight)


def kernel(values, offsets, position_weight):
    n = values.shape[0]
    b = offsets.shape[0]
    return _position_weights(values, offsets, position_weight, n, b)


# speedup vs baseline: 1.0743x; 1.0743x over previous
import functools

import jax
import jax.numpy as jnp
from jax import lax
from jax.experimental import pallas as pl
from jax.experimental.pallas import tpu as pltpu
from jax.experimental.pallas import tpu_sc as plsc

_NUM_CORES = 1
_NUM_SUBCORES = 16
_LANES = 16
_NW = _NUM_CORES * _NUM_SUBCORES
_ROW = 128


@functools.partial(jax.jit, static_argnames=("n", "b"))
def _position_weights(values, offsets, position_weight, n, b):
    chunk = n // _NW
    vecs = chunk // _LANES
    rows = chunk // _ROW
    vecs_per_row = _ROW // _LANES
    mesh = plsc.VectorSubcoreMesh(
        core_axis_name="c", subcore_axis_name="s", num_cores=_NUM_CORES
    )

    @functools.partial(
        pl.kernel,
        mesh=mesh,
        out_type=(
            jax.ShapeDtypeStruct((n,), values.dtype),
            jax.ShapeDtypeStruct((b,), offsets.dtype),
            jax.ShapeDtypeStruct((n,), jnp.float32),
        ),
        compiler_params=pltpu.CompilerParams(needs_layout_passes=False),
        scratch_types=[
            pltpu.VMEM((_LANES,), jnp.int32),
            pltpu.VMEM((n,), jnp.float32),
            pltpu.VMEM((chunk,), jnp.int32),
            pltpu.VMEM((chunk,), jnp.float32),
            pltpu.SemaphoreType.DMA,
            pltpu.SemaphoreType.DMA,
            pltpu.SemaphoreType.DMA,
        ],
    )
    def body(values_hbm, offs_hbm, pw_hbm, vout_hbm, oout_hbm, out_hbm,
             offs_v, pw_v, seq_arr, out_v, tsem, osem, psem):
        wid = lax.axis_index("s") * _NUM_CORES + lax.axis_index("c")
        base = wid * chunk
        table_dma = pltpu.async_copy(pw_hbm, pw_v, tsem)
        pass_dma = pltpu.async_copy(
            values_hbm.at[pl.ds(base, chunk)], vout_hbm.at[pl.ds(base, chunk)], psem
        )

        offs_pass_dma = pltpu.async_copy(offs_hbm, oout_hbm, psem)
        pltpu.sync_copy(offs_hbm.at[pl.ds(0, _LANES)], offs_v)

        offs_vec = offs_v[...]
        zero = jnp.zeros((_LANES,), jnp.int32)

        def zero_step(v, carry):
            seq_arr[pl.ds(v * _LANES, _LANES)] = zero
            return carry

        lax.fori_loop(0, vecs, zero_step, 0)
        carry0 = jnp.max(jnp.where(offs_vec <= base, offs_vec, 0))
        in_tile = (offs_vec > base) & (offs_vec < base + chunk)
        plsc.store_scatter(seq_arr, [offs_vec - base], offs_vec, mask=in_tile)

        lane = lax.iota(jnp.int32, _LANES)

        def seq_step(v, carry):
            start = v * _LANES
            off = jnp.maximum(plsc.cummax(seq_arr[pl.ds(start, _LANES)]), carry)
            seq_arr[pl.ds(start, _LANES)] = lane + (base + start) - off
            return off[_LANES - 1]

        lax.fori_loop(0, vecs, seq_step, carry0)

        table_dma.wait()
        out_dmas = []
        for r in range(rows):
            for v in range(vecs_per_row):
                start = r * _ROW + v * _LANES
                out_v[start:start + _LANES] = plsc.load_gather(
                    pw_v, [seq_arr[start:start + _LANES]]
                )
            out_dmas.append(
                pltpu.async_copy(
                    out_v.at[pl.ds(r * _ROW, _ROW)],
                    out_hbm.at[pl.ds(base + r * _ROW, _ROW)],
                    osem,
                )
            )
        for d in out_dmas:
            d.wait()
        pass_dma.wait()
        offs_pass_dma.wait()

    return body(values, offsets, position_weight)


def kernel(values, offsets, position_weight):
    n = values.shape[0]
    b = offsets.shape[0]
    return _position_weights(values, offsets, position_weight, n, b)
